# Initial kernel scaffold; baseline (speedup 1.0000x reference)
import jax
import jax.numpy as jnp
from jax.experimental import pallas as pl


def kernel(x, edge_index, w1, b1, w2, b2, W, bias):
    raise NotImplementedError("write your pallas kernel here")


# speedup vs baseline: 9.1116x; 9.1116x over previous
import functools

import jax
import jax.numpy as jnp
from jax import lax
from jax.experimental import pallas as pl
from jax.experimental.pallas import tpu as pltpu
from jax.experimental.pallas import tpu_sc as plsc

N = 10000
P = 3
F = 128
FH = F // 2
NPAD = 10240
NODE = NPAD // 16
E2PAD = 335872
MAIN_CH = 128
DEG_CH = 512
N_MAIN = E2PAD // (16 * MAIN_CH)
N_DEG = E2PAD // (16 * DEG_CH)


def _sc_body(row_hbm, col_hbm, coors_hbm, feats_hbm, params_hbm,
             ones_hbm, zro_hbm, zrod_hbm, out_hbm,
             coors_v, dinv_v, msg_v, row_v, rowg_v, col_v, colL_v, s_v,
             dgb_v, dinvs_v, params_v, ones_v,
             acc_sh, deg_sh, dinv_sh, sem):
    ci = lax.axis_index("c")
    ti = lax.axis_index("s")

    pltpu.sync_copy(zro_hbm, acc_sh.at[pl.ds(ti * NODE, NODE)])
    pltpu.sync_copy(zrod_hbm, deg_sh.at[pl.ds(ti * NODE, NODE)])
    pltpu.sync_copy(ones_hbm, ones_v)
    pltpu.sync_copy(coors_hbm, coors_v)
    pltpu.sync_copy(params_hbm, params_v)
    plsc.subcore_barrier()

    def deg_body(i, carry):
        base = (ti * N_DEG + i) * DEG_CH
        pltpu.sync_copy(col_hbm.at[pl.ds(base, DEG_CH)], colL_v)
        pltpu.sync_copy(ones_v, deg_sh.at[colL_v], add=True)
        return carry

    lax.fori_loop(0, N_DEG, deg_body, 0)
    plsc.subcore_barrier()

    pltpu.sync_copy(deg_sh.at[pl.ds(ti * NODE, NODE)], dgb_v)

    def dinv_body(g, carry):
        d = dgb_v[pl.ds(g * 16, 16)]
        xi = lax.bitcast_convert_type(d, jnp.int32)
        yi = jnp.int32(0x5F3759DF) - (xi >> 1)
        y = lax.bitcast_convert_type(yi, jnp.float32)
        for _ in range(3):
            y = y * (1.5 - 0.5 * d * y * y)
        dinvs_v[pl.ds(g * 16, 16)] = y
        return carry

    lax.fori_loop(0, NODE // 16, dinv_body, 0)
    pltpu.sync_copy(dinvs_v, dinv_sh.at[pl.ds(ti * NODE, NODE)])
    plsc.subcore_barrier()
    pltpu.sync_copy(dinv_sh, dinv_v)

    a_vec = params_v[pl.ds(0, 16)]
    c_vec = params_v[pl.ds(16, 16)]
    goff = ci * NPAD

    def main_body(i, carry):
        base = (ti * N_MAIN + i) * MAIN_CH
        pltpu.sync_copy(row_hbm.at[pl.ds(base, MAIN_CH)], row_v)
        pltpu.sync_copy(col_hbm.at[pl.ds(base, MAIN_CH)], col_v)
        for g in range(MAIN_CH // 16):
            rowg_v[pl.ds(g * 16, 16)] = row_v[pl.ds(g * 16, 16)] + goff
        pltpu.async_copy(feats_hbm.at[rowg_v], msg_v, sem).wait()
        for g in range(MAIN_CH // 16):
            ir = row_v[pl.ds(g * 16, 16)]
            ic = col_v[pl.ds(g * 16, 16)]
            i4r = ir * 4
            i4c = ic * 4
            dx = (plsc.load_gather(coors_v, [i4r])
                  - plsc.load_gather(coors_v, [i4c]))
            dy = (plsc.load_gather(coors_v, [i4r + 1])
                  - plsc.load_gather(coors_v, [i4c + 1]))
            dz = (plsc.load_gather(coors_v, [i4r + 2])
                  - plsc.load_gather(coors_v, [i4c + 2]))
            rd = dx * dx + dy * dy + dz * dz
            z = a_vec * rd + c_vec
            tmp = 1.0 / (1.0 + jnp.exp(-z))
            dr = plsc.load_gather(dinv_v, [ir])
            dc = plsc.load_gather(dinv_v, [ic])
            s_v[pl.ds(g * 16, 16)] = tmp * dr * dc
        jz = row_v[pl.ds(0, 16)] >> 31
        for j in range(MAIN_CH):
            sb = plsc.load_gather(s_v, [jz + j])
            for f in range(FH // 16):
                msg_v[j, pl.ds(f * 16, 16)] = msg_v[j, pl.ds(f * 16, 16)] * sb
        pltpu.sync_copy(msg_v, acc_sh.at[col_v], add=True)
        return carry

    lax.fori_loop(0, N_MAIN, main_body, 0)
    plsc.subcore_barrier()

    pltpu.sync_copy(acc_sh.at[pl.ds(ti * NODE, NODE)],
                    out_hbm.at[ci, pl.ds(ti * NODE, NODE)])


_sc_kernel = functools.partial(
    pl.kernel,
    _sc_body,
    out_type=jax.ShapeDtypeStruct((2, NPAD, FH), jnp.float32),
    mesh=plsc.VectorSubcoreMesh(core_axis_name="c", subcore_axis_name="s"),
    compiler_params=pltpu.CompilerParams(needs_layout_passes=False,
                                         use_tc_tiling_on_sc=False),
    scratch_types=[
        pltpu.VMEM((NPAD * 4,), jnp.float32),
        pltpu.VMEM((NPAD,), jnp.float32),
        pltpu.VMEM((MAIN_CH, FH), jnp.float32),
        pltpu.VMEM((MAIN_CH,), jnp.int32),
        pltpu.VMEM((MAIN_CH,), jnp.int32),
        pltpu.VMEM((MAIN_CH,), jnp.int32),
        pltpu.VMEM((DEG_CH,), jnp.int32),
        pltpu.VMEM((MAIN_CH,), jnp.float32),
        pltpu.VMEM((NODE,), jnp.float32),
        pltpu.VMEM((NODE,), jnp.float32),
        pltpu.VMEM((32,), jnp.float32),
        pltpu.VMEM((DEG_CH,), jnp.float32),
        pltpu.VMEM_SHARED((NPAD, FH), jnp.float32),
        pltpu.VMEM_SHARED((NPAD,), jnp.float32),
        pltpu.VMEM_SHARED((NPAD,), jnp.float32),
        pltpu.SemaphoreType.DMA,
    ],
)()


def _tc_body(p0_ref, p1_ref, w_ref, b_ref, o_ref):
    o_ref[...] = (
        jnp.dot(p0_ref[...], w_ref[:FH, :], preferred_element_type=jnp.float32)
        + jnp.dot(p1_ref[...], w_ref[FH:, :], preferred_element_type=jnp.float32)
        + b_ref[...])


def kernel(x, edge_index, w1, b1, w2, b2, W, bias):
    n = x.shape[0]
    e = edge_index.shape[1]
    coors = x[:, :P]
    feats = x[:, P:]

    a = (w1 @ w2)[0, 0]
    c = (b1 @ w2 + b2)[0]
    params = jnp.concatenate([jnp.full((16,), a, jnp.float32),
                              jnp.full((16,), c, jnp.float32)])

    loop = jnp.arange(n, dtype=jnp.int32)
    pad = E2PAD - (e + n)
    rowp = jnp.concatenate([edge_index[0], loop, jnp.full((pad,), n, jnp.int32)])
    colp = jnp.concatenate([edge_index[1], loop, jnp.full((pad,), n, jnp.int32)])

    coors_pad = jnp.zeros((NPAD, 4), jnp.float32).at[:n, :P].set(coors)
    feats_pad = jnp.zeros((2 * NPAD, FH), jnp.float32)
    feats_pad = feats_pad.at[:n, :].set(feats[:, :FH])
    feats_pad = feats_pad.at[NPAD:NPAD + n, :].set(feats[:, FH:])

    partials = _sc_kernel(rowp, colp, coors_pad.reshape(-1), feats_pad, params,
                          jnp.ones((DEG_CH,), jnp.float32),
                          jnp.zeros((NODE, FH), jnp.float32),
                          jnp.zeros((NODE,), jnp.float32))

    blk = 1000
    hidden = pl.pallas_call(
        _tc_body,
        grid=(n // blk,),
        in_specs=[
            pl.BlockSpec((blk, FH), lambda i: (i, 0)),
            pl.BlockSpec((blk, FH), lambda i: (i, 0)),
            pl.BlockSpec((F, F), lambda i: (0, 0)),
            pl.BlockSpec((1, F), lambda i: (0, 0)),
        ],
        out_specs=pl.BlockSpec((blk, F), lambda i: (i, 0)),
        out_shape=jax.ShapeDtypeStruct((n, F), jnp.float32),
    )(partials[0, :n], partials[1, :n], W, bias[None, :])

    return jnp.concatenate([coors, hidden], axis=-1)


# speedup vs baseline: 11.2557x; 1.2353x over previous
import functools

import jax
import jax.numpy as jnp
from jax import lax
from jax.experimental import pallas as pl
from jax.experimental.pallas import tpu as pltpu
from jax.experimental.pallas import tpu_sc as plsc

N = 10000
P = 3
F = 128
FH = F // 2
NPAD = 10240
NODE = NPAD // 16
E2PAD = 335872
MAIN_CH = 64
NCH = E2PAD // MAIN_CH
N_MAIN = NCH // 16
DEG_CH = 2624
N_DEG = E2PAD // (16 * DEG_CH)


def _sc_body(rc_hbm, col_hbm, coors_hbm, feats_hbm, params_hbm,
             ones_hbm, zro_hbm, zrod_hbm, out_hbm,
             coors_v, dinv_v, msg0_v, msg1_v, rc0_v, rc1_v, rowg0_v, rowg1_v,
             colL_v, s_v, dgb_v, dinvs_v, params_v, ones_v,
             acc_sh, deg_sh, dinv_sh, gsem0, gsem1, isem0, isem1, sem):
    ci = lax.axis_index("c")
    ti = lax.axis_index("s")

    pltpu.sync_copy(zro_hbm, acc_sh.at[pl.ds(ti * NODE, NODE)])
    pltpu.sync_copy(zrod_hbm, deg_sh.at[pl.ds(ti * NODE, NODE)])
    pltpu.sync_copy(ones_hbm, ones_v)
    pltpu.sync_copy(coors_hbm, coors_v)
    pltpu.sync_copy(params_hbm, params_v)
    plsc.subcore_barrier()

    def deg_body(i, carry):
        base = (ti * N_DEG + i) * DEG_CH
        pltpu.sync_copy(col_hbm.at[pl.ds(base, DEG_CH)], colL_v)
        pltpu.sync_copy(ones_v, deg_sh.at[colL_v], add=True)
        return carry

    lax.fori_loop(0, N_DEG, deg_body, 0)
    plsc.subcore_barrier()

    pltpu.sync_copy(deg_sh.at[pl.ds(ti * NODE, NODE)], dgb_v)

    def dinv_body(g, carry):
        d = dgb_v[pl.ds(g * 16, 16)]
        xi = lax.bitcast_convert_type(d, jnp.int32)
        yi = jnp.int32(0x5F3759DF) - (xi >> 1)
        y = lax.bitcast_convert_type(yi, jnp.float32)
        for _ in range(3):
            y = y * (1.5 - 0.5 * d * y * y)
        dinvs_v[pl.ds(g * 16, 16)] = y
        return carry

    lax.fori_loop(0, NODE // 16, dinv_body, 0)
    pltpu.sync_copy(dinvs_v, dinv_sh.at[pl.ds(ti * NODE, NODE)])
    plsc.subcore_barrier()
    pltpu.sync_copy(dinv_sh, dinv_v)

    a_vec = params_v[pl.ds(0, 16)]
    c_vec = params_v[pl.ds(16, 16)]
    goff = ci * NPAD
    chunk0 = ti * N_MAIN

    def proc(k, rc_c, rowg_c, msg_c, gsem_c, isem_c,
             rc_n, rowg_n, msg_n, gsem_n, isem_n):
        pltpu.make_async_copy(rc_hbm.at[chunk0], rc_n, isem_n).wait()
        for g in range(MAIN_CH // 16):
            rowg_n[pl.ds(g * 16, 16)] = rc_n[0, pl.ds(g * 16, 16)] + goff
        gather_n = pltpu.async_copy(feats_hbm.at[rowg_n], msg_n, gsem_n)
        pltpu.make_async_copy(feats_hbm.at[rowg_c], msg_c, gsem_c).wait()
        for g in range(MAIN_CH // 16):
            ir = rc_c[0, pl.ds(g * 16, 16)]
            ic = rc_c[1, pl.ds(g * 16, 16)]
            i4r = ir * 4
            i4c = ic * 4
            dx = (plsc.load_gather(coors_v, [i4r])
                  - plsc.load_gather(coors_v, [i4c]))
            dy = (plsc.load_gather(coors_v, [i4r + 1])
                  - plsc.load_gather(coors_v, [i4c + 1]))
            dz = (plsc.load_gather(coors_v, [i4r + 2])
                  - plsc.load_gather(coors_v, [i4c + 2]))
            rd = dx * dx + dy * dy + dz * dz
            z = a_vec * rd + c_vec
            tmp = 1.0 / (1.0 + jnp.exp(-z))
            dr = plsc.load_gather(dinv_v, [ir])
            dc = plsc.load_gather(dinv_v, [ic])
            s_v[pl.ds(g * 16, 16)] = tmp * dr * dc
        jz = rc_c[0, pl.ds(0, 16)] >> 31
        for j in range(MAIN_CH):
            sb = plsc.load_gather(s_v, [jz + j])
            for f in range(FH // 16):
                msg_c[j, pl.ds(f * 16, 16)] = msg_c[j, pl.ds(f * 16, 16)] * sb
        pltpu.sync_copy(msg_c, acc_sh.at[rc_c.at[1]], add=True)
        pltpu.async_copy(rc_hbm.at[chunk0 + k + 2], rc_c, isem_c)
        return gather_n

    pltpu.sync_copy(rc_hbm.at[chunk0], rc0_v)
    for g in range(MAIN_CH // 16):
        rowg0_v[pl.ds(g * 16, 16)] = rc0_v[0, pl.ds(g * 16, 16)] + goff
    pltpu.async_copy(feats_hbm.at[rowg0_v], msg0_v, gsem0)
    pltpu.async_copy(rc_hbm.at[chunk0 + 1], rc1_v, isem1)

    def main_body(j, carry):
        k = j * 2
        proc(k, rc0_v, rowg0_v, msg0_v, gsem0, isem0,
             rc1_v, rowg1_v, msg1_v, gsem1, isem1)
        proc(k + 1, rc1_v, rowg1_v, msg1_v, gsem1, isem1,
             rc0_v, rowg0_v, msg0_v, gsem0, isem0)
        return carry

    lax.fori_loop(0, N_MAIN // 2, main_body, 0)
    pltpu.make_async_copy(feats_hbm.at[rowg0_v], msg0_v, gsem0).wait()
    pltpu.make_async_copy(rc_hbm.at[chunk0], rc1_v, isem1).wait()
    plsc.subcore_barrier()

    pltpu.sync_copy(acc_sh.at[pl.ds(ti * NODE, NODE)],
                    out_hbm.at[ci, pl.ds(ti * NODE, NODE)])


_sc_kernel = functools.partial(
    pl.kernel,
    _sc_body,
    out_type=jax.ShapeDtypeStruct((2, NPAD, FH), jnp.float32),
    mesh=plsc.VectorSubcoreMesh(core_axis_name="c", subcore_axis_name="s"),
    compiler_params=pltpu.CompilerParams(needs_layout_passes=False,
                                         use_tc_tiling_on_sc=False),
    scratch_types=[
        pltpu.VMEM((NPAD * 4,), jnp.float32),
        pltpu.VMEM((NPAD,), jnp.float32),
        pltpu.VMEM((MAIN_CH, FH), jnp.float32),
        pltpu.VMEM((MAIN_CH, FH), jnp.float32),
        pltpu.VMEM((2, MAIN_CH), jnp.int32),
        pltpu.VMEM((2, MAIN_CH), jnp.int32),
        pltpu.VMEM((MAIN_CH,), jnp.int32),
        pltpu.VMEM((MAIN_CH,), jnp.int32),
        pltpu.VMEM((DEG_CH,), jnp.int32),
        pltpu.VMEM((MAIN_CH,), jnp.float32),
        pltpu.VMEM((NODE,), jnp.float32),
        pltpu.VMEM((NODE,), jnp.float32),
        pltpu.VMEM((32,), jnp.float32),
        pltpu.VMEM((DEG_CH,), jnp.float32),
        pltpu.VMEM_SHARED((NPAD, FH), jnp.float32),
        pltpu.VMEM_SHARED((NPAD,), jnp.float32),
        pltpu.VMEM_SHARED((NPAD,), jnp.float32),
        pltpu.SemaphoreType.DMA,
        pltpu.SemaphoreType.DMA,
        pltpu.SemaphoreType.DMA,
        pltpu.SemaphoreType.DMA,
        pltpu.SemaphoreType.DMA,
    ],
)()


def _tc_body(p0_ref, p1_ref, w_ref, b_ref, o_ref):
    o_ref[...] = (
        jnp.dot(p0_ref[...], w_ref[:FH, :], preferred_element_type=jnp.float32)
        + jnp.dot(p1_ref[...], w_ref[FH:, :], preferred_element_type=jnp.float32)
        + b_ref[...])


def kernel(x, edge_index, w1, b1, w2, b2, W, bias):
    n = x.shape[0]
    e = edge_index.shape[1]
    coors = x[:, :P]
    feats = x[:, P:]

    a = (w1 @ w2)[0, 0]
    c = (b1 @ w2 + b2)[0]
    params = jnp.concatenate([jnp.full((16,), a, jnp.float32),
                              jnp.full((16,), c, jnp.float32)])

    loop = jnp.arange(n, dtype=jnp.int32)
    pad = E2PAD - (e + n)
    rowp = jnp.concatenate([edge_index[0], loop, jnp.full((pad,), n, jnp.int32)])
    colp = jnp.concatenate([edge_index[1], loop, jnp.full((pad,), n, jnp.int32)])
    rc = jnp.stack([rowp.reshape(NCH, MAIN_CH), colp.reshape(NCH, MAIN_CH)],
                   axis=1)
    rc = jnp.concatenate(
        [rc, jnp.full((2, 2, MAIN_CH), n, jnp.int32)], axis=0)

    coors_pad = jnp.zeros((NPAD, 4), jnp.float32).at[:n, :P].set(coors)
    feats_pad = jnp.zeros((2 * NPAD, FH), jnp.float32)
    feats_pad = feats_pad.at[:n, :].set(feats[:, :FH])
    feats_pad = feats_pad.at[NPAD:NPAD + n, :].set(feats[:, FH:])

    partials = _sc_kernel(rc, colp, coors_pad.reshape(-1), feats_pad, params,
                          jnp.ones((DEG_CH,), jnp.float32),
                          jnp.zeros((NODE, FH), jnp.float32),
                          jnp.zeros((NODE,), jnp.float32))

    blk = 1000
    hidden = pl.pallas_call(
        _tc_body,
        grid=(n // blk,),
        in_specs=[
            pl.BlockSpec((blk, FH), lambda i: (i, 0)),
            pl.BlockSpec((blk, FH), lambda i: (i, 0)),
            pl.BlockSpec((F, F), lambda i: (0, 0)),
            pl.BlockSpec((1, F), lambda i: (0, 0)),
        ],
        out_specs=pl.BlockSpec((blk, F), lambda i: (i, 0)),
        out_shape=jax.ShapeDtypeStruct((n, F), jnp.float32),
    )(partials[0, :n], partials[1, :n], W, bias[None, :])

    return jnp.concatenate([coors, hidden], axis=-1)


# speedup vs baseline: 12.0478x; 1.0704x over previous
import functools

import jax
import jax.numpy as jnp
from jax import lax
from jax.experimental import pallas as pl
from jax.experimental.pallas import tpu as pltpu
from jax.experimental.pallas import tpu_sc as plsc

N = 10000
P = 3
F = 128
FH = F // 2
NPAD = 10240
NODE = NPAD // 16
E2PAD = 335872
MAIN_CH = 64
NCH = E2PAD // MAIN_CH
N_MAIN = NCH // 16
DEG_CH = 2624
N_DEG = E2PAD // (16 * DEG_CH)


def _sc_body(rc_hbm, col_hbm, coors_hbm, feats_hbm, params_hbm,
             ones_hbm, zro_hbm, zrod_hbm, out_hbm,
             coors_v, dinv_v, msg0_v, msg1_v, rc0_v, rc1_v, rowg0_v, rowg1_v,
             col0_v, col1_v, colL_v, s_v, dgb_v, dinvs_v, params_v, ones_v,
             acc_sh, deg_sh, dinv_sh,
             gsem0, gsem1, isem0, isem1, ssem0, ssem1):
    ci = lax.axis_index("c")
    ti = lax.axis_index("s")

    pltpu.sync_copy(zro_hbm, acc_sh.at[pl.ds(ti * NODE, NODE)])
    pltpu.sync_copy(zrod_hbm, deg_sh.at[pl.ds(ti * NODE, NODE)])
    pltpu.sync_copy(ones_hbm, ones_v)
    pltpu.sync_copy(coors_hbm, coors_v)
    pltpu.sync_copy(params_hbm, params_v)
    plsc.subcore_barrier()

    def deg_body(i, carry):
        base = (ti * N_DEG + i) * DEG_CH
        pltpu.sync_copy(col_hbm.at[pl.ds(base, DEG_CH)], colL_v)
        pltpu.sync_copy(ones_v, deg_sh.at[colL_v], add=True)
        return carry

    lax.fori_loop(0, N_DEG, deg_body, 0)
    plsc.subcore_barrier()

    pltpu.sync_copy(deg_sh.at[pl.ds(ti * NODE, NODE)], dgb_v)

    def dinv_body(g, carry):
        d = dgb_v[pl.ds(g * 16, 16)]
        xi = lax.bitcast_convert_type(d, jnp.int32)
        yi = jnp.int32(0x5F3759DF) - (xi >> 1)
        y = lax.bitcast_convert_type(yi, jnp.float32)
        for _ in range(3):
            y = y * (1.5 - 0.5 * d * y * y)
        dinvs_v[pl.ds(g * 16, 16)] = y
        return carry

    lax.fori_loop(0, NODE // 16, dinv_body, 0)
    pltpu.sync_copy(dinvs_v, dinv_sh.at[pl.ds(ti * NODE, NODE)])
    plsc.subcore_barrier()
    pltpu.sync_copy(dinv_sh, dinv_v)

    a_vec = params_v[pl.ds(0, 16)]
    c_vec = params_v[pl.ds(16, 16)]
    goff = ci * NPAD
    chunk0 = ti * N_MAIN

    def proc(k, rc_c, rowg_c, col_c, msg_c, gsem_c, isem_c, ssem_c,
             rc_n, rowg_n, col_n, msg_n, gsem_n, isem_n, ssem_n):
        pltpu.make_async_copy(rc_hbm.at[chunk0], rc_n, isem_n).wait()
        for g in range(MAIN_CH // 16):
            rowg_n[pl.ds(g * 16, 16)] = rc_n[0, pl.ds(g * 16, 16)] + goff
        pltpu.make_async_copy(msg_n, acc_sh.at[col_n], ssem_n).wait()
        pltpu.async_copy(feats_hbm.at[rowg_n], msg_n, gsem_n)
        pltpu.make_async_copy(feats_hbm.at[rowg_c], msg_c, gsem_c).wait()
        for g in range(MAIN_CH // 16):
            ir = rc_c[0, pl.ds(g * 16, 16)]
            ic = rc_c[1, pl.ds(g * 16, 16)]
            col_c[pl.ds(g * 16, 16)] = ic
            i4r = ir * 4
            i4c = ic * 4
            dx = (plsc.load_gather(coors_v, [i4r])
                  - plsc.load_gather(coors_v, [i4c]))
            dy = (plsc.load_gather(coors_v, [i4r + 1])
                  - plsc.load_gather(coors_v, [i4c + 1]))
            dz = (plsc.load_gather(coors_v, [i4r + 2])
                  - plsc.load_gather(coors_v, [i4c + 2]))
            rd = dx * dx + dy * dy + dz * dz
            z = a_vec * rd + c_vec
            tmp = 1.0 / (1.0 + jnp.exp(-z))
            dr = plsc.load_gather(dinv_v, [ir])
            dc = plsc.load_gather(dinv_v, [ic])
            s_v[pl.ds(g * 16, 16)] = tmp * dr * dc
        jz = rc_c[0, pl.ds(0, 16)] >> 31
        for j in range(MAIN_CH):
            sb = plsc.load_gather(s_v, [jz + j])
            for f in range(FH // 16):
                msg_c[j, pl.ds(f * 16, 16)] = msg_c[j, pl.ds(f * 16, 16)] * sb
        pltpu.async_copy(msg_c, acc_sh.at[col_c], ssem_c, add=True)
        pltpu.async_copy(rc_hbm.at[chunk0 + k + 2], rc_c, isem_c)

    pltpu.sync_copy(rc_hbm.at[chunk0], rc0_v)
    for g in range(MAIN_CH // 16):
        rowg0_v[pl.ds(g * 16, 16)] = rc0_v[0, pl.ds(g * 16, 16)] + goff
        col1_v[pl.ds(g * 16, 16)] = rc0_v[1, pl.ds(g * 16, 16)]
    pltpu.async_copy(feats_hbm.at[rowg0_v], msg0_v, gsem0)
    pltpu.sync_copy(zro_hbm.at[pl.ds(0, MAIN_CH)], msg1_v)
    pltpu.async_copy(msg1_v, acc_sh.at[col1_v], ssem1, add=True)
    pltpu.async_copy(rc_hbm.at[chunk0 + 1], rc1_v, isem1)

    def main_body(j, carry):
        k = j * 2
        proc(k, rc0_v, rowg0_v, col0_v, msg0_v, gsem0, isem0, ssem0,
             rc1_v, rowg1_v, col1_v, msg1_v, gsem1, isem1, ssem1)
        proc(k + 1, rc1_v, rowg1_v, col1_v, msg1_v, gsem1, isem1, ssem1,
             rc0_v, rowg0_v, col0_v, msg0_v, gsem0, isem0, ssem0)
        return carry

    lax.fori_loop(0, N_MAIN // 2, main_body, 0)
    pltpu.make_async_copy(feats_hbm.at[rowg0_v], msg0_v, gsem0).wait()
    pltpu.make_async_copy(rc_hbm.at[chunk0], rc1_v, isem1).wait()
    pltpu.make_async_copy(msg1_v, acc_sh.at[col1_v], ssem1).wait()
    plsc.subcore_barrier()

    pltpu.sync_copy(acc_sh.at[pl.ds(ti * NODE, NODE)],
                    out_hbm.at[ci, pl.ds(ti * NODE, NODE)])


_sc_kernel = functools.partial(
    pl.kernel,
    _sc_body,
    out_type=jax.ShapeDtypeStruct((2, NPAD, FH), jnp.float32),
    mesh=plsc.VectorSubcoreMesh(core_axis_name="c", subcore_axis_name="s"),
    compiler_params=pltpu.CompilerParams(needs_layout_passes=False,
                                         use_tc_tiling_on_sc=False),
    scratch_types=[
        pltpu.VMEM((NPAD * 4,), jnp.float32),
        pltpu.VMEM((NPAD,), jnp.float32),
        pltpu.VMEM((MAIN_CH, FH), jnp.float32),
        pltpu.VMEM((MAIN_CH, FH), jnp.float32),
        pltpu.VMEM((2, MAIN_CH), jnp.int32),
        pltpu.VMEM((2, MAIN_CH), jnp.int32),
        pltpu.VMEM((MAIN_CH,), jnp.int32),
        pltpu.VMEM((MAIN_CH,), jnp.int32),
        pltpu.VMEM((MAIN_CH,), jnp.int32),
        pltpu.VMEM((MAIN_CH,), jnp.int32),
        pltpu.VMEM((DEG_CH,), jnp.int32),
        pltpu.VMEM((MAIN_CH,), jnp.float32),
        pltpu.VMEM((NODE,), jnp.float32),
        pltpu.VMEM((NODE,), jnp.float32),
        pltpu.VMEM((32,), jnp.float32),
        pltpu.VMEM((DEG_CH,), jnp.float32),
        pltpu.VMEM_SHARED((NPAD, FH), jnp.float32),
        pltpu.VMEM_SHARED((NPAD,), jnp.float32),
        pltpu.VMEM_SHARED((NPAD,), jnp.float32),
        pltpu.SemaphoreType.DMA,
        pltpu.SemaphoreType.DMA,
        pltpu.SemaphoreType.DMA,
        pltpu.SemaphoreType.DMA,
        pltpu.SemaphoreType.DMA,
        pltpu.SemaphoreType.DMA,
    ],
)()


def _tc_body(p0_ref, p1_ref, w_ref, b_ref, o_ref):
    o_ref[...] = (
        jnp.dot(p0_ref[...], w_ref[:FH, :], preferred_element_type=jnp.float32)
        + jnp.dot(p1_ref[...], w_ref[FH:, :], preferred_element_type=jnp.float32)
        + b_ref[...])


def kernel(x, edge_index, w1, b1, w2, b2, W, bias):
    n = x.shape[0]
    e = edge_index.shape[1]
    coors = x[:, :P]
    feats = x[:, P:]

    a = (w1 @ w2)[0, 0]
    c = (b1 @ w2 + b2)[0]
    params = jnp.concatenate([jnp.full((16,), a, jnp.float32),
                              jnp.full((16,), c, jnp.float32)])

    loop = jnp.arange(n, dtype=jnp.int32)
    pad = E2PAD - (e + n)
    rowp = jnp.concatenate([edge_index[0], loop, jnp.full((pad,), n, jnp.int32)])
    colp = jnp.concatenate([edge_index[1], loop, jnp.full((pad,), n, jnp.int32)])
    rc = jnp.stack([rowp.reshape(NCH, MAIN_CH), colp.reshape(NCH, MAIN_CH)],
                   axis=1)
    rc = jnp.concatenate(
        [rc, jnp.full((2, 2, MAIN_CH), n, jnp.int32)], axis=0)

    coors_pad = jnp.zeros((NPAD, 4), jnp.float32).at[:n, :P].set(coors)
    feats_pad = jnp.zeros((2 * NPAD, FH), jnp.float32)
    feats_pad = feats_pad.at[:n, :].set(feats[:, :FH])
    feats_pad = feats_pad.at[NPAD:NPAD + n, :].set(feats[:, FH:])

    partials = _sc_kernel(rc, colp, coors_pad.reshape(-1), feats_pad, params,
                          jnp.ones((DEG_CH,), jnp.float32),
                          jnp.zeros((NODE, FH), jnp.float32),
                          jnp.zeros((NODE,), jnp.float32))

    blk = 1000
    hidden = pl.pallas_call(
        _tc_body,
        grid=(n // blk,),
        in_specs=[
            pl.BlockSpec((blk, FH), lambda i: (i, 0)),
            pl.BlockSpec((blk, FH), lambda i: (i, 0)),
            pl.BlockSpec((F, F), lambda i: (0, 0)),
            pl.BlockSpec((1, F), lambda i: (0, 0)),
        ],
        out_specs=pl.BlockSpec((blk, F), lambda i: (i, 0)),
        out_shape=jax.ShapeDtypeStruct((n, F), jnp.float32),
    )(partials[0, :n], partials[1, :n], W, bias[None, :])

    return jnp.concatenate([coors, hidden], axis=-1)
